# Initial kernel scaffold; baseline (speedup 1.0000x reference)
import jax
import jax.numpy as jnp
from jax.experimental import pallas as pl


def kernel(h, x, edge_index, W_emb, b_emb, W_filt, b_filt, iW1, ib1, iW2, ib2, mW1, mb1, mW2, mb2, mWmu):
    raise NotImplementedError("write your pallas kernel here")


# speedup vs baseline: 8.1701x; 8.1701x over previous
import functools
import math

import jax
import jax.numpy as jnp
import numpy as np
from jax import lax
from jax.experimental import pallas as pl
from jax.experimental.pallas import tpu as pltpu
from jax.experimental.pallas import tpu_sc as plsc

_NC = 2
_NS = 16
_NW = _NC * _NS
_CHUNK = 128

_CUT = 5.0
_EPS = 1e-8
_NUM_RBF = 50
_RBF_PAD = 64


@functools.lru_cache(maxsize=None)
def _sc_gather(n_rows, d, ep):
    per_w = ep // _NW
    n_chunks = per_w // _CHUNK
    mesh = plsc.VectorSubcoreMesh(core_axis_name="c", subcore_axis_name="s", num_cores=_NC, num_subcores=_NS)

    def body(table_hbm, idx_hbm, out_hbm, idx_v, rows_v, sem):
        wid = lax.axis_index("s") * _NC + lax.axis_index("c")
        base = wid * per_w

        def step(k, carry):
            off = base + k * _CHUNK
            pltpu.sync_copy(idx_hbm.at[pl.ds(off, _CHUNK)], idx_v)
            pltpu.async_copy(table_hbm.at[idx_v], rows_v, sem).wait()
            pltpu.sync_copy(rows_v, out_hbm.at[pl.ds(off, _CHUNK)])
            return carry

        lax.fori_loop(0, n_chunks, step, 0)

    return pl.kernel(
        body,
        out_type=jax.ShapeDtypeStruct((ep, d), jnp.float32),
        mesh=mesh,
        scratch_types=[
            pltpu.VMEM((_CHUNK,), jnp.int32),
            pltpu.VMEM((_CHUNK, d), jnp.float32),
            pltpu.SemaphoreType.DMA,
        ],
    )


@functools.lru_cache(maxsize=None)
def _sc_scatter_add(n_rows, ep):
    per_sc = ep // _NC
    per_w = per_sc // _NS
    n_chunks = per_w // _CHUNK
    rpt = n_rows // _NS
    zr = 128
    assert n_rows % (_NS * zr) == 0
    mesh = plsc.VectorSubcoreMesh(core_axis_name="c", subcore_axis_name="s", num_cores=_NC, num_subcores=_NS)

    def body(vals_hbm, idx_hbm, out_hbm, idx_v, vals_v, zbuf, acc):
        c = lax.axis_index("c")
        s = lax.axis_index("s")

        def zstep(i, carry):
            r = i // 8
            col = lax.rem(i, 8)
            zbuf[r, pl.ds(col * 16, 16)] = jnp.zeros((16,), jnp.float32)
            return carry

        lax.fori_loop(0, zr * 8, zstep, 0)

        def zcopy(j, carry):
            pltpu.sync_copy(zbuf, acc.at[pl.ds(s * rpt + j * zr, zr)])
            return carry

        lax.fori_loop(0, rpt // zr, zcopy, 0)
        plsc.subcore_barrier()

        base = c * per_sc + s * per_w

        def step(k, carry):
            off = base + k * _CHUNK
            pltpu.sync_copy(idx_hbm.at[pl.ds(off, _CHUNK)], idx_v)
            pltpu.sync_copy(vals_hbm.at[pl.ds(off, _CHUNK)], vals_v)
            pltpu.sync_copy(vals_v, acc.at[idx_v], add=True)
            return carry

        lax.fori_loop(0, n_chunks, step, 0)
        plsc.subcore_barrier()

        def ostep(j, carry):
            row = s * rpt + j * zr
            pltpu.sync_copy(acc.at[pl.ds(row, zr)], zbuf)
            pltpu.sync_copy(zbuf, out_hbm.at[c, pl.ds(row, zr)])
            return carry

        lax.fori_loop(0, rpt // zr, ostep, 0)

    return pl.kernel(
        body,
        out_type=jax.ShapeDtypeStruct((_NC, n_rows, 128), jnp.float32),
        mesh=mesh,
        scratch_types=[
            pltpu.VMEM((_CHUNK,), jnp.int32),
            pltpu.VMEM((_CHUNK, 128), jnp.float32),
            pltpu.VMEM((zr, 128), jnp.float32),
            pltpu.VMEM_SHARED((n_rows, 128), jnp.float32),
        ],
    )


def _silu(t):
    return t / (1.0 + jnp.exp(-t))


def _tc_filters(xi, xj, wf, bf, n_edges, be=640):
    ep = xi.shape[0]
    grid = ep // be
    delta = _CUT / (_NUM_RBF - 1)
    coeff = -0.5 / float(delta * delta)

    def body(xi_ref, xj_ref, wf_ref, bf_ref, filt_ref, dir_ref):
        rbf_id = lax.broadcasted_iota(jnp.int32, (1, _RBF_PAD), 1)
        offs_c = jnp.where(rbf_id < _NUM_RBF,
                           rbf_id.astype(jnp.float32) * delta, 1.0e3)
        i = pl.program_id(0)
        gid = i * be + lax.broadcasted_iota(jnp.int32, (be, 1), 0)
        valid = (gid < n_edges).astype(jnp.float32)
        r = xj_ref[:, :8] - xi_ref[:, :8]
        col = lax.broadcasted_iota(jnp.int32, (be, 8), 1)
        e_x = (col == 0).astype(jnp.float32)
        r = jnp.where(gid < n_edges, r, e_x)
        d2 = jnp.sum(r[:, :3] * r[:, :3], axis=1, keepdims=True)
        d = jnp.sqrt(d2)
        dirv = r[:, :8] / d
        phi = jnp.exp(coeff * (d - offs_c) ** 2)
        fcut = 0.5 * (jnp.cos(d * (math.pi / _CUT)) + 1.0)
        fcut = fcut * (d < _CUT).astype(jnp.float32) * valid
        filt = (jnp.dot(phi, wf_ref[...],
                        preferred_element_type=jnp.float32) + bf_ref[...]) * fcut
        filt_ref[...] = filt
        dir_ref[...] = dirv

    return pl.pallas_call(
        body,
        grid=(grid,),
        in_specs=[
            pl.BlockSpec((be, 128), lambda i: (i, 0)),
            pl.BlockSpec((be, 128), lambda i: (i, 0)),
            pl.BlockSpec((_RBF_PAD, 768), lambda i: (0, 0)),
            pl.BlockSpec((1, 768), lambda i: (0, 0)),
        ],
        out_specs=[
            pl.BlockSpec((be, 768), lambda i: (i, 0)),
            pl.BlockSpec((be, 8), lambda i: (i, 0)),
        ],
        out_shape=[
            jax.ShapeDtypeStruct((ep, 768), jnp.float32),
            jax.ShapeDtypeStruct((ep, 8), jnp.float32),
        ],
    )(xi, xj, wf, bf)


def _tc_embed(h, w, b, bn=1000):
    n = h.shape[0]

    def body(h_ref, w_ref, b_ref, o_ref):
        o_ref[...] = jnp.dot(h_ref[...], w_ref[...],
                             preferred_element_type=jnp.float32) + b_ref[...]

    return pl.pallas_call(
        body,
        grid=(n // bn,),
        in_specs=[
            pl.BlockSpec((bn, 128), lambda i: (i, 0)),
            pl.BlockSpec((128, 128), lambda i: (0, 0)),
            pl.BlockSpec((1, 128), lambda i: (0, 0)),
        ],
        out_specs=pl.BlockSpec((bn, 128), lambda i: (i, 0)),
        out_shape=jax.ShapeDtypeStruct((n, 128), jnp.float32),
    )(h, w, b)


def _tc_interaction_mlp(q, w1, b1, w2, b2, bn=1000):
    n = q.shape[0]

    def body(q_ref, w1_ref, b1_ref, w2_ref, b2_ref, o_ref):
        t = _silu(jnp.dot(q_ref[...], w1_ref[...],
                          preferred_element_type=jnp.float32) + b1_ref[...])
        o_ref[...] = jnp.dot(t, w2_ref[...],
                             preferred_element_type=jnp.float32) + b2_ref[...]

    return pl.pallas_call(
        body,
        grid=(n // bn,),
        in_specs=[
            pl.BlockSpec((bn, 128), lambda i: (i, 0)),
            pl.BlockSpec((128, 128), lambda i: (0, 0)),
            pl.BlockSpec((1, 128), lambda i: (0, 0)),
            pl.BlockSpec((128, 384), lambda i: (0, 0)),
            pl.BlockSpec((1, 384), lambda i: (0, 0)),
        ],
        out_specs=pl.BlockSpec((bn, 384), lambda i: (i, 0)),
        out_shape=jax.ShapeDtypeStruct((n, 384), jnp.float32),
    )(q, w1, b1, w2, b2)


def _tc_edge_products(filt, xj, dirp, muj, be=640):
    ep = filt.shape[0]
    with_mu = muj is not None

    def body(*refs):
        if with_mu:
            filt_ref, xj_ref, dir_ref, muj_ref, dq_ref, dmu_ref = refs
        else:
            filt_ref, xj_ref, dir_ref, dq_ref, dmu_ref = refs
        xf = filt_ref[...] * xj_ref[...]
        dq_ref[...] = xf[:, :128]
        dmur = xf[:, 128:256]
        dmum = xf[:, 256:384]
        parts = []
        for dax in range(3):
            t = dmur * dir_ref[:, dax:dax + 1]
            if with_mu:
                t = t + dmum * muj_ref[:, dax * 128:(dax + 1) * 128]
            parts.append(t)
        dmu_ref[...] = jnp.stack(parts, axis=0)

    in_specs = [
        pl.BlockSpec((be, 384), lambda i: (i, 0)),
        pl.BlockSpec((be, 384), lambda i: (i, 0)),
        pl.BlockSpec((be, 8), lambda i: (i, 0)),
    ]
    args = [filt, xj, dirp]
    if with_mu:
        in_specs.append(pl.BlockSpec((be, 384), lambda i: (i, 0)))
        args.append(muj)

    return pl.pallas_call(
        body,
        grid=(ep // be,),
        in_specs=in_specs,
        out_specs=[
            pl.BlockSpec((be, 128), lambda i: (i, 0)),
            pl.BlockSpec((3, be, 128), lambda i: (0, i, 0)),
        ],
        out_shape=[
            jax.ShapeDtypeStruct((ep, 128), jnp.float32),
            jax.ShapeDtypeStruct((3, ep, 128), jnp.float32),
        ],
    )(*args)


def _tc_update_mixing(q, mu, dqp, dmup0, dmup1, dmup2,
                      wmu, w1, b1, w2, b2, bn=1000):
    n = q.shape[0]
    with_mu = mu is not None

    def body(*refs):
        if with_mu:
            (q_ref, mu_ref, dqp_ref, d0_ref, d1_ref, d2_ref,
             wmu_ref, w1_ref, b1_ref, w2_ref, b2_ref, qo_ref, muo_ref) = refs
        else:
            (q_ref, dqp_ref, d0_ref, d1_ref, d2_ref,
             wmu_ref, w1_ref, b1_ref, w2_ref, b2_ref, qo_ref, muo_ref) = refs
        q1 = q_ref[...] + dqp_ref[0] + dqp_ref[1]
        mu1 = []
        for dax, dref in enumerate((d0_ref, d1_ref, d2_ref)):
            m = dref[0] + dref[1]
            if with_mu:
                m = m + mu_ref[:, dax * 128:(dax + 1) * 128]
            mu1.append(m)
        wmu_v = wmu_ref[...]
        mix = [jnp.dot(m, wmu_v, preferred_element_type=jnp.float32)
               for m in mu1]
        muv = [mx[:, :128] for mx in mix]
        muw = [mx[:, 128:] for mx in mix]
        muvn = jnp.sqrt(muv[0] * muv[0] + muv[1] * muv[1]
                        + muv[2] * muv[2] + _EPS)
        ctx = jnp.concatenate([q1, muvn], axis=1)
        t = _silu(jnp.dot(ctx, w1_ref[...],
                          preferred_element_type=jnp.float32) + b1_ref[...])
        xm = jnp.dot(t, w2_ref[...],
                     preferred_element_type=jnp.float32) + b2_ref[...]
        dq_i = xm[:, :128]
        dmu_i = xm[:, 128:256]
        dqmu_i = xm[:, 256:384]
        dot = muv[0] * muw[0] + muv[1] * muw[1] + muv[2] * muw[2]
        qo_ref[...] = q1 + dq_i + dqmu_i * dot
        muo_ref[...] = jnp.concatenate(
            [mu1[dax] + dmu_i * muw[dax] for dax in range(3)], axis=1)

    in_specs = [pl.BlockSpec((bn, 128), lambda i: (i, 0))]
    args = [q]
    if with_mu:
        in_specs.append(pl.BlockSpec((bn, 384), lambda i: (i, 0)))
        args.append(mu)
    for a in (dqp, dmup0, dmup1, dmup2):
        in_specs.append(pl.BlockSpec((2, bn, 128), lambda i: (0, i, 0)))
        args.append(a)
    in_specs += [
        pl.BlockSpec((128, 256), lambda i: (0, 0)),
        pl.BlockSpec((256, 128), lambda i: (0, 0)),
        pl.BlockSpec((1, 128), lambda i: (0, 0)),
        pl.BlockSpec((128, 384), lambda i: (0, 0)),
        pl.BlockSpec((1, 384), lambda i: (0, 0)),
    ]
    args += [wmu, w1, b1, w2, b2]

    return pl.pallas_call(
        body,
        grid=(n // bn,),
        in_specs=in_specs,
        out_specs=[
            pl.BlockSpec((bn, 128), lambda i: (i, 0)),
            pl.BlockSpec((bn, 384), lambda i: (i, 0)),
        ],
        out_shape=[
            jax.ShapeDtypeStruct((n, 128), jnp.float32),
            jax.ShapeDtypeStruct((n, 384), jnp.float32),
        ],
    )(*args)


def kernel(h, x, edge_index, W_emb, b_emb, W_filt, b_filt,
           iW1, ib1, iW2, ib2, mW1, mb1, mW2, mb2, mWmu):
    n, in_dim = h.shape
    e = edge_index.shape[1]
    f = W_emb.shape[1]
    num_l = iW1.shape[0]
    ep = ((e + 4095) // 4096) * 4096
    n_acc = ((n + 2047) // 2048) * 2048

    idx_i = edge_index[0]
    idx_j = edge_index[1]
    pad_e = ep - e
    idx_i_p = jnp.pad(idx_i, (0, pad_e))
    idx_j_p = jnp.pad(idx_j, (0, pad_e))

    x128 = jnp.pad(x, ((0, 0), (0, 128 - x.shape[1])))
    h_pad = jnp.pad(h, ((0, 0), (0, 128 - in_dim)))
    wemb_pad = jnp.pad(W_emb, ((0, 128 - in_dim), (0, 0)))
    wf_pad = jnp.pad(W_filt, ((0, _RBF_PAD - W_filt.shape[0]), (0, 0)))

    xi = _sc_gather(n, 128, ep)(x128, idx_i_p)
    xj = _sc_gather(n, 128, ep)(x128, idx_j_p)
    filt_all, dirp = _tc_filters(xi, xj, wf_pad, b_filt[None, :], e)

    q = _tc_embed(h_pad, wemb_pad, b_emb[None, :])
    mu = None

    for l in range(num_l):
        filt_l = lax.slice_in_dim(filt_all, l * 3 * f, (l + 1) * 3 * f, axis=1)
        xc = _tc_interaction_mlp(q, iW1[l], ib1[l][None, :],
                                 iW2[l], ib2[l][None, :])
        xcj = _sc_gather(n, 384, ep)(xc, idx_j_p)
        muj = None if mu is None else _sc_gather(n, 384, ep)(mu, idx_j_p)
        dq_e, dmu_e = _tc_edge_products(filt_l, xcj, dirp, muj)
        dqp = _sc_scatter_add(n_acc, ep)(dq_e, idx_i_p)
        dmup = [_sc_scatter_add(n_acc, ep)(dmu_e[dax], idx_i_p)
                for dax in range(3)]
        q, mu = _tc_update_mixing(q, mu, dqp, dmup[0], dmup[1], dmup[2],
                                  mWmu[l], mW1[l], mb1[l][None, :],
                                  mW2[l], mb2[l][None, :])

    return q, mu.reshape(n, 3, f)


# speedup vs baseline: 9.7001x; 1.1873x over previous
import functools
import math

import jax
import jax.numpy as jnp
import numpy as np
from jax import lax
from jax.experimental import pallas as pl
from jax.experimental.pallas import tpu as pltpu
from jax.experimental.pallas import tpu_sc as plsc

_NC = 2
_NS = 16
_NW = _NC * _NS
_CHUNK = 128

_CUT = 5.0
_EPS = 1e-8
_NUM_RBF = 50
_RBF_PAD = 64


@functools.lru_cache(maxsize=None)
def _sc_gather(n_rows, d, ep):
    per_w = ep // _NW
    n_chunks = per_w // _CHUNK
    assert n_chunks % 2 == 0
    mesh = plsc.VectorSubcoreMesh(core_axis_name="c", subcore_axis_name="s", num_cores=_NC, num_subcores=_NS)

    def body(table_hbm, idx_hbm2, out_hbm, idx_all, rows, g0, g1, o0, o1):
        wid = lax.axis_index("s") * _NC + lax.axis_index("c")
        cbase = pl.multiple_of(wid * n_chunks, 8)
        pltpu.sync_copy(idx_hbm2.at[pl.ds(cbase, n_chunks)], idx_all)
        gsem = (g0, g1)
        osem = (o0, o1)

        def gstart(ch, b):
            pltpu.async_copy(table_hbm.at[idx_all.at[ch]], rows.at[b], gsem[b])

        def gwait(b):
            pltpu.make_async_copy(out_hbm.at[pl.ds(0, _CHUNK)], rows.at[b],
                                  gsem[b]).wait()

        def ostart(ch, b):
            off = (cbase + ch) * _CHUNK
            pltpu.async_copy(rows.at[b], out_hbm.at[pl.ds(off, _CHUNK)],
                             osem[b])

        def owait(b):
            pltpu.make_async_copy(rows.at[b], out_hbm.at[pl.ds(0, _CHUNK)],
                                  osem[b]).wait()

        gstart(0, 0)
        gstart(1, 1)

        def pair(j, carry):
            for b in (0, 1):
                ch = 2 * j + b
                gwait(b)
                ostart(ch, b)
                owait(b)

                @pl.when(ch + 2 < n_chunks)
                def _():
                    gstart(ch + 2, b)
            return carry

        lax.fori_loop(0, n_chunks // 2, pair, 0)

    return pl.kernel(
        body,
        out_type=jax.ShapeDtypeStruct((ep, d), jnp.float32),
        mesh=mesh,
        scratch_types=[
            pltpu.VMEM((n_chunks, _CHUNK), jnp.int32),
            pltpu.VMEM((2, _CHUNK, d), jnp.float32),
            pltpu.SemaphoreType.DMA,
            pltpu.SemaphoreType.DMA,
            pltpu.SemaphoreType.DMA,
            pltpu.SemaphoreType.DMA,
        ],
    )


@functools.lru_cache(maxsize=None)
def _sc_scatter_add(n_rows, ep):
    per_sc = ep // _NC
    per_w = per_sc // _NS
    n_chunks = per_w // _CHUNK
    rpt = n_rows // _NS
    zr = 32
    assert n_rows % (_NS * zr) == 0
    mesh = plsc.VectorSubcoreMesh(core_axis_name="c", subcore_axis_name="s", num_cores=_NC, num_subcores=_NS)

    def body(vals_hbm, idx_hbm2, out_hbm, idx_all, vals, zbuf, acc,
             v0, v1, s0, s1):
        c = lax.axis_index("c")
        s = lax.axis_index("s")

        def zstep(i, carry):
            r = i // 8
            col = lax.rem(i, 8)
            zbuf[r, pl.ds(col * 16, 16)] = jnp.zeros((16,), jnp.float32)
            return carry

        lax.fori_loop(0, zr * 8, zstep, 0)

        def zcopy(j, carry):
            pltpu.sync_copy(zbuf, acc.at[pl.ds(s * rpt + j * zr, zr)])
            return carry

        lax.fori_loop(0, rpt // zr, zcopy, 0)
        plsc.subcore_barrier()

        cbase = pl.multiple_of((c * per_sc + s * per_w) // _CHUNK, 8)
        pltpu.sync_copy(idx_hbm2.at[pl.ds(cbase, n_chunks)], idx_all)
        vsem = (v0, v1)
        ssem = (s0, s1)

        def vstart(ch, b):
            off = (cbase + ch) * _CHUNK
            pltpu.async_copy(vals_hbm.at[pl.ds(off, _CHUNK)], vals.at[b],
                             vsem[b])

        def vwait(b):
            pltpu.make_async_copy(vals_hbm.at[pl.ds(0, _CHUNK)], vals.at[b],
                                  vsem[b]).wait()

        def sstart(ch, b):
            pltpu.async_copy(vals.at[b], acc.at[idx_all.at[ch]], ssem[b],
                             add=True)

        def swait(b):
            pltpu.make_async_copy(vals_hbm.at[pl.ds(0, _CHUNK)], vals.at[b],
                                  ssem[b]).wait()

        vstart(0, 0)
        vstart(1, 1)

        def pair(j, carry):
            for b in (0, 1):
                ch = 2 * j + b
                vwait(b)
                sstart(ch, b)
                swait(b)

                @pl.when(ch + 2 < n_chunks)
                def _():
                    vstart(ch + 2, b)
            return carry

        lax.fori_loop(0, n_chunks // 2, pair, 0)
        plsc.subcore_barrier()

        def ostep(j, carry):
            row = s * rpt + j * zr
            pltpu.sync_copy(acc.at[pl.ds(row, zr)], zbuf)
            pltpu.sync_copy(zbuf, out_hbm.at[c, pl.ds(row, zr)])
            return carry

        lax.fori_loop(0, rpt // zr, ostep, 0)

    return pl.kernel(
        body,
        out_type=jax.ShapeDtypeStruct((_NC, n_rows, 128), jnp.float32),
        mesh=mesh,
        scratch_types=[
            pltpu.VMEM((n_chunks, _CHUNK), jnp.int32),
            pltpu.VMEM((2, _CHUNK, 128), jnp.float32),
            pltpu.VMEM((zr, 128), jnp.float32),
            pltpu.VMEM_SHARED((n_rows, 128), jnp.float32),
            pltpu.SemaphoreType.DMA,
            pltpu.SemaphoreType.DMA,
            pltpu.SemaphoreType.DMA,
            pltpu.SemaphoreType.DMA,
        ],
    )


def _silu(t):
    return t / (1.0 + jnp.exp(-t))


def _tc_filters(xi, xj, wf, bf, n_edges, be=640):
    ep = xi.shape[0]
    grid = ep // be
    delta = _CUT / (_NUM_RBF - 1)
    coeff = -0.5 / float(delta * delta)

    def body(xi_ref, xj_ref, wf_ref, bf_ref, filt_ref, dir_ref):
        rbf_id = lax.broadcasted_iota(jnp.int32, (1, _RBF_PAD), 1)
        offs_c = jnp.where(rbf_id < _NUM_RBF,
                           rbf_id.astype(jnp.float32) * delta, 1.0e3)
        i = pl.program_id(0)
        gid = i * be + lax.broadcasted_iota(jnp.int32, (be, 1), 0)
        valid = (gid < n_edges).astype(jnp.float32)
        r = xj_ref[:, :8] - xi_ref[:, :8]
        col = lax.broadcasted_iota(jnp.int32, (be, 8), 1)
        e_x = (col == 0).astype(jnp.float32)
        r = jnp.where(gid < n_edges, r, e_x)
        d2 = jnp.sum(r[:, :3] * r[:, :3], axis=1, keepdims=True)
        d = jnp.sqrt(d2)
        dirv = r[:, :8] / d
        phi = jnp.exp(coeff * (d - offs_c) ** 2)
        fcut = 0.5 * (jnp.cos(d * (math.pi / _CUT)) + 1.0)
        fcut = fcut * (d < _CUT).astype(jnp.float32) * valid
        filt = (jnp.dot(phi, wf_ref[...],
                        preferred_element_type=jnp.float32) + bf_ref[...]) * fcut
        filt_ref[...] = filt
        dir_ref[...] = dirv

    return pl.pallas_call(
        body,
        grid=(grid,),
        in_specs=[
            pl.BlockSpec((be, 128), lambda i: (i, 0)),
            pl.BlockSpec((be, 128), lambda i: (i, 0)),
            pl.BlockSpec((_RBF_PAD, 768), lambda i: (0, 0)),
            pl.BlockSpec((1, 768), lambda i: (0, 0)),
        ],
        out_specs=[
            pl.BlockSpec((be, 768), lambda i: (i, 0)),
            pl.BlockSpec((be, 8), lambda i: (i, 0)),
        ],
        out_shape=[
            jax.ShapeDtypeStruct((ep, 768), jnp.float32),
            jax.ShapeDtypeStruct((ep, 8), jnp.float32),
        ],
    )(xi, xj, wf, bf)


def _tc_embed(h, w, b, bn=1000):
    n = h.shape[0]

    def body(h_ref, w_ref, b_ref, o_ref):
        o_ref[...] = jnp.dot(h_ref[...], w_ref[...],
                             preferred_element_type=jnp.float32) + b_ref[...]

    return pl.pallas_call(
        body,
        grid=(n // bn,),
        in_specs=[
            pl.BlockSpec((bn, 128), lambda i: (i, 0)),
            pl.BlockSpec((128, 128), lambda i: (0, 0)),
            pl.BlockSpec((1, 128), lambda i: (0, 0)),
        ],
        out_specs=pl.BlockSpec((bn, 128), lambda i: (i, 0)),
        out_shape=jax.ShapeDtypeStruct((n, 128), jnp.float32),
    )(h, w, b)


def _tc_interaction_mlp(q, w1, b1, w2, b2, bn=1000):
    n = q.shape[0]

    def body(q_ref, w1_ref, b1_ref, w2_ref, b2_ref, o_ref):
        t = _silu(jnp.dot(q_ref[...], w1_ref[...],
                          preferred_element_type=jnp.float32) + b1_ref[...])
        o_ref[...] = jnp.dot(t, w2_ref[...],
                             preferred_element_type=jnp.float32) + b2_ref[...]

    return pl.pallas_call(
        body,
        grid=(n // bn,),
        in_specs=[
            pl.BlockSpec((bn, 128), lambda i: (i, 0)),
            pl.BlockSpec((128, 128), lambda i: (0, 0)),
            pl.BlockSpec((1, 128), lambda i: (0, 0)),
            pl.BlockSpec((128, 384), lambda i: (0, 0)),
            pl.BlockSpec((1, 384), lambda i: (0, 0)),
        ],
        out_specs=pl.BlockSpec((bn, 384), lambda i: (i, 0)),
        out_shape=jax.ShapeDtypeStruct((n, 384), jnp.float32),
    )(q, w1, b1, w2, b2)


def _tc_edge_products(filt, xj, dirp, muj, be=640):
    ep = filt.shape[0]
    with_mu = muj is not None

    def body(*refs):
        if with_mu:
            filt_ref, xj_ref, dir_ref, muj_ref, dq_ref, dmu_ref = refs
        else:
            filt_ref, xj_ref, dir_ref, dq_ref, dmu_ref = refs
        xf = filt_ref[...] * xj_ref[...]
        dq_ref[...] = xf[:, :128]
        dmur = xf[:, 128:256]
        dmum = xf[:, 256:384]
        parts = []
        for dax in range(3):
            t = dmur * dir_ref[:, dax:dax + 1]
            if with_mu:
                t = t + dmum * muj_ref[:, dax * 128:(dax + 1) * 128]
            parts.append(t)
        dmu_ref[...] = jnp.stack(parts, axis=0)

    in_specs = [
        pl.BlockSpec((be, 384), lambda i: (i, 0)),
        pl.BlockSpec((be, 384), lambda i: (i, 0)),
        pl.BlockSpec((be, 8), lambda i: (i, 0)),
    ]
    args = [filt, xj, dirp]
    if with_mu:
        in_specs.append(pl.BlockSpec((be, 384), lambda i: (i, 0)))
        args.append(muj)

    return pl.pallas_call(
        body,
        grid=(ep // be,),
        in_specs=in_specs,
        out_specs=[
            pl.BlockSpec((be, 128), lambda i: (i, 0)),
            pl.BlockSpec((3, be, 128), lambda i: (0, i, 0)),
        ],
        out_shape=[
            jax.ShapeDtypeStruct((ep, 128), jnp.float32),
            jax.ShapeDtypeStruct((3, ep, 128), jnp.float32),
        ],
    )(*args)


def _tc_update_mixing(q, mu, dqp, dmup0, dmup1, dmup2,
                      wmu, w1, b1, w2, b2, bn=1000):
    n = q.shape[0]
    with_mu = mu is not None

    def body(*refs):
        if with_mu:
            (q_ref, mu_ref, dqp_ref, d0_ref, d1_ref, d2_ref,
             wmu_ref, w1_ref, b1_ref, w2_ref, b2_ref, qo_ref, muo_ref) = refs
        else:
            (q_ref, dqp_ref, d0_ref, d1_ref, d2_ref,
             wmu_ref, w1_ref, b1_ref, w2_ref, b2_ref, qo_ref, muo_ref) = refs
        q1 = q_ref[...] + dqp_ref[0] + dqp_ref[1]
        mu1 = []
        for dax, dref in enumerate((d0_ref, d1_ref, d2_ref)):
            m = dref[0] + dref[1]
            if with_mu:
                m = m + mu_ref[:, dax * 128:(dax + 1) * 128]
            mu1.append(m)
        wmu_v = wmu_ref[...]
        mix = [jnp.dot(m, wmu_v, preferred_element_type=jnp.float32)
               for m in mu1]
        muv = [mx[:, :128] for mx in mix]
        muw = [mx[:, 128:] for mx in mix]
        muvn = jnp.sqrt(muv[0] * muv[0] + muv[1] * muv[1]
                        + muv[2] * muv[2] + _EPS)
        ctx = jnp.concatenate([q1, muvn], axis=1)
        t = _silu(jnp.dot(ctx, w1_ref[...],
                          preferred_element_type=jnp.float32) + b1_ref[...])
        xm = jnp.dot(t, w2_ref[...],
                     preferred_element_type=jnp.float32) + b2_ref[...]
        dq_i = xm[:, :128]
        dmu_i = xm[:, 128:256]
        dqmu_i = xm[:, 256:384]
        dot = muv[0] * muw[0] + muv[1] * muw[1] + muv[2] * muw[2]
        qo_ref[...] = q1 + dq_i + dqmu_i * dot
        muo_ref[...] = jnp.concatenate(
            [mu1[dax] + dmu_i * muw[dax] for dax in range(3)], axis=1)

    in_specs = [pl.BlockSpec((bn, 128), lambda i: (i, 0))]
    args = [q]
    if with_mu:
        in_specs.append(pl.BlockSpec((bn, 384), lambda i: (i, 0)))
        args.append(mu)
    for a in (dqp, dmup0, dmup1, dmup2):
        in_specs.append(pl.BlockSpec((2, bn, 128), lambda i: (0, i, 0)))
        args.append(a)
    in_specs += [
        pl.BlockSpec((128, 256), lambda i: (0, 0)),
        pl.BlockSpec((256, 128), lambda i: (0, 0)),
        pl.BlockSpec((1, 128), lambda i: (0, 0)),
        pl.BlockSpec((128, 384), lambda i: (0, 0)),
        pl.BlockSpec((1, 384), lambda i: (0, 0)),
    ]
    args += [wmu, w1, b1, w2, b2]

    return pl.pallas_call(
        body,
        grid=(n // bn,),
        in_specs=in_specs,
        out_specs=[
            pl.BlockSpec((bn, 128), lambda i: (i, 0)),
            pl.BlockSpec((bn, 384), lambda i: (i, 0)),
        ],
        out_shape=[
            jax.ShapeDtypeStruct((n, 128), jnp.float32),
            jax.ShapeDtypeStruct((n, 384), jnp.float32),
        ],
    )(*args)


def kernel(h, x, edge_index, W_emb, b_emb, W_filt, b_filt,
           iW1, ib1, iW2, ib2, mW1, mb1, mW2, mb2, mWmu):
    n, in_dim = h.shape
    e = edge_index.shape[1]
    f = W_emb.shape[1]
    num_l = iW1.shape[0]
    ep = ((e + 4095) // 4096) * 4096
    n_acc = ((n + 2047) // 2048) * 2048

    idx_i = edge_index[0]
    idx_j = edge_index[1]
    pad_e = ep - e
    idx_i_p = jnp.pad(idx_i, (0, pad_e)).reshape(-1, _CHUNK)
    idx_j_p = jnp.pad(idx_j, (0, pad_e)).reshape(-1, _CHUNK)

    x128 = jnp.pad(x, ((0, 0), (0, 128 - x.shape[1])))
    h_pad = jnp.pad(h, ((0, 0), (0, 128 - in_dim)))
    wemb_pad = jnp.pad(W_emb, ((0, 128 - in_dim), (0, 0)))
    wf_pad = jnp.pad(W_filt, ((0, _RBF_PAD - W_filt.shape[0]), (0, 0)))

    xi = _sc_gather(n, 128, ep)(x128, idx_i_p)
    xj = _sc_gather(n, 128, ep)(x128, idx_j_p)
    filt_all, dirp = _tc_filters(xi, xj, wf_pad, b_filt[None, :], e)

    q = _tc_embed(h_pad, wemb_pad, b_emb[None, :])
    mu = None

    for l in range(num_l):
        filt_l = lax.slice_in_dim(filt_all, l * 3 * f, (l + 1) * 3 * f, axis=1)
        xc = _tc_interaction_mlp(q, iW1[l], ib1[l][None, :],
                                 iW2[l], ib2[l][None, :])
        xcj = _sc_gather(n, 384, ep)(xc, idx_j_p)
        muj = None if mu is None else _sc_gather(n, 384, ep)(mu, idx_j_p)
        dq_e, dmu_e = _tc_edge_products(filt_l, xcj, dirp, muj)
        dqp = _sc_scatter_add(n_acc, ep)(dq_e, idx_i_p)
        dmup = [_sc_scatter_add(n_acc, ep)(dmu_e[dax], idx_i_p)
                for dax in range(3)]
        q, mu = _tc_update_mixing(q, mu, dqp, dmup[0], dmup[1], dmup[2],
                                  mWmu[l], mW1[l], mb1[l][None, :],
                                  mW2[l], mb2[l][None, :])

    return q, mu.reshape(n, 3, f)
